# baseline (speedup 1.0000x reference)
import functools

import jax
import jax.numpy as jnp
from jax import lax
from jax.experimental import pallas as pl
from jax.experimental.pallas import tpu as pltpu
from jax.experimental.pallas import tpu_sc as plsc

_CH = 8
_NW = 32
_B = 4096
_LD = 12800
_RPW = _LD // _NW
_NCH = _RPW // _CH


def _sc_kernel(x_hbm, pe_hbm, o_hbm, buf0, buf1, pe_v,
               si0, si1, so0, so1):
    wid = lax.axis_index("s") * 2 + lax.axis_index("c")
    base = wid * _RPW
    pltpu.sync_copy(pe_hbm.at[pl.ds(base, _RPW)], pe_v)

    bufs = (buf0, buf1)
    sin = (si0, si1)
    sout = (so0, so1)

    def in_copy(c):
        s = c % 2
        return pltpu.make_async_copy(
            x_hbm.at[pl.ds(base + c * _CH, _CH)], bufs[s], sin[s])

    def out_copy(c):
        s = c % 2
        return pltpu.make_async_copy(
            bufs[s], o_hbm.at[pl.ds(base + c * _CH, _CH)], sout[s])

    in_copy(0).start()
    in_copy(1).start()
    for c in range(_NCH):
        s = c % 2
        in_copy(c).wait()
        buf = bufs[s]

        def row_body(r, _):
            idx = jnp.full((16,), c * _CH + r, jnp.int32)
            pe_splat = plsc.load_gather(pe_v, [idx])

            def k_body(k, _):
                sl = pl.ds(k * 16, 16)
                buf[r, sl] = buf[r, sl] + pe_splat
                return 0

            lax.fori_loop(0, _B // 16, k_body, 0)
            return 0

        lax.fori_loop(0, _CH, row_body, 0)
        out_copy(c).start()
        if c + 2 < _NCH:
            out_copy(c).wait()
            in_copy(c + 2).start()
    out_copy(_NCH - 2).wait()
    out_copy(_NCH - 1).wait()


def kernel(x, encoding):
    B, L, D = x.shape
    LD = L * D
    x2 = x.transpose(1, 2, 0).reshape(LD, B)
    pe = encoding[:L].reshape(LD)
    mesh = plsc.VectorSubcoreMesh(core_axis_name="c", subcore_axis_name="s")
    f = functools.partial(
        pl.kernel,
        mesh=mesh,
        out_type=jax.ShapeDtypeStruct((LD, B), x.dtype),
        scratch_types=[
            pltpu.VMEM((_CH, _B), jnp.float32),
            pltpu.VMEM((_CH, _B), jnp.float32),
            pltpu.VMEM((_RPW,), jnp.float32),
            pltpu.SemaphoreType.DMA,
            pltpu.SemaphoreType.DMA,
            pltpu.SemaphoreType.DMA,
            pltpu.SemaphoreType.DMA,
        ],
        compiler_params=pltpu.CompilerParams(needs_layout_passes=False),
    )(_sc_kernel)
    out = f(x2, pe)
    return out.reshape(L, D, B).transpose(2, 0, 1)


# speedup vs baseline: 4.7197x; 4.7197x over previous
import jax
import jax.numpy as jnp
from jax.experimental import pallas as pl
from jax.experimental.pallas import tpu as pltpu

_CL = 8
_K = 3


def _make_body(L, num_chunks):
    def body(x_hbm, et_vmem, o_hbm, in_buf, out_buf, in_sem, out_sem):
        def start_in(c):
            s = c % _K
            pltpu.make_async_copy(
                x_hbm.at[pl.ds(c * _CL, _CL)], in_buf.at[s], in_sem.at[s]
            ).start()

        for c in range(min(_K, num_chunks)):
            start_in(c)
        for c in range(num_chunks):
            s = c % _K
            pltpu.make_async_copy(
                x_hbm.at[pl.ds(c * _CL, _CL)], in_buf.at[s], in_sem.at[s]
            ).wait()
            if c >= _K:
                pltpu.make_async_copy(
                    out_buf.at[s],
                    o_hbm.at[pl.ds((c - _K) * _CL, _CL)],
                    out_sem.at[s],
                ).wait()
            for j in range(_CL):
                l = c * _CL + j
                out_buf[s, j] = in_buf[s, j] + et_vmem[:, l:l + 1]
            pltpu.make_async_copy(
                out_buf.at[s], o_hbm.at[pl.ds(c * _CL, _CL)], out_sem.at[s]
            ).start()
            if c + _K < num_chunks:
                start_in(c + _K)
        for c in range(max(0, num_chunks - _K), num_chunks):
            s = c % _K
            pltpu.make_async_copy(
                out_buf.at[s], o_hbm.at[pl.ds(c * _CL, _CL)], out_sem.at[s]
            ).wait()

    return body


def kernel(x, encoding):
    B, L, D = x.shape
    num_chunks = L // _CL
    x3 = x.transpose(1, 2, 0)
    et = encoding.T
    out = pl.pallas_call(
        _make_body(L, num_chunks),
        in_specs=[
            pl.BlockSpec(memory_space=pl.ANY),
            pl.BlockSpec(memory_space=pltpu.VMEM),
        ],
        out_specs=pl.BlockSpec(memory_space=pl.ANY),
        out_shape=jax.ShapeDtypeStruct((L, D, B), x.dtype),
        scratch_shapes=[
            pltpu.VMEM((_K, _CL, D, B), x.dtype),
            pltpu.VMEM((_K, _CL, D, B), x.dtype),
            pltpu.SemaphoreType.DMA((_K,)),
            pltpu.SemaphoreType.DMA((_K,)),
        ],
        compiler_params=pltpu.CompilerParams(
            vmem_limit_bytes=110 * 1024 * 1024,
        ),
    )(x3, et)
    return out.transpose(2, 0, 1)
